# baseline (speedup 1.0000x reference)
import jax
import jax.numpy as jnp
from jax import lax
from jax.experimental import pallas as pl
from jax.experimental.pallas import tpu as pltpu
from jax.experimental.pallas import tpu_sc as plsc

ENTITY_NUM = 1000000
REL_NUM = 1000
DE = 64
DR = 32
B = 16384

NC = 2
NS = 16
L = 16
NW = NC * NS
BPW = B // NW
G = L
NG = BPW // G
NSLOT = 2


def _rsqrt_nr(x):
    i = lax.bitcast_convert_type(x, jnp.int32)
    y = lax.bitcast_convert_type(jnp.int32(0x5F3759DF) - (i >> 1), jnp.float32)
    for _ in range(3):
        y = y * (1.5 - 0.5 * x * y * y)
    return y


@jax.jit
def _tripletembed(ent, rel, relm, h, r, t):
    mesh = plsc.VectorSubcoreMesh(core_axis_name="c", subcore_axis_name="s",
                                  num_cores=NC, num_subcores=NS)

    def body(ent, rel, relm, h, r, t, hp, reo, tp,
             hidx, ridx, tidx,
             heb0, heb1, teb0, teb1, mb0, mb1, rpb0, rpb1,
             hps, res, tps,
             semh0, semh1, semt0, semt1, semm0, semm1, semr0, semr1):
        c = lax.axis_index("c")
        s = lax.axis_index("s")
        wid = s * NC + c
        wbase = wid * BPW

        pltpu.sync_copy(h.at[pl.ds(wbase, BPW)], hidx)
        pltpu.sync_copy(r.at[pl.ds(wbase, BPW)], ridx)
        pltpu.sync_copy(t.at[pl.ds(wbase, BPW)], tidx)

        hebs = (heb0, heb1)
        tebs = (teb0, teb1)
        mbs = (mb0, mb1)
        rpbs = (rpb0, rpb1)
        semh = (semh0, semh1)
        semt = (semt0, semt1)
        semm = (semm0, semm1)
        semr = (semr0, semr1)

        lanes = lax.iota(jnp.int32, L)
        zeros = jnp.zeros((L,), jnp.float32)
        ones = jnp.ones((L,), jnp.float32)

        def fire(g, sl):
            gbase = g * G
            hvec = hidx[pl.ds(gbase, G)]
            tvec = tidx[pl.ds(gbase, G)]
            rvec = ridx[pl.ds(gbase, G)]
            pltpu.async_copy(ent.at[hvec], hebs[sl], semh[sl])
            pltpu.async_copy(ent.at[tvec], tebs[sl], semt[sl])
            pltpu.async_copy(relm.at[rvec], mbs[sl], semm[sl])
            pltpu.async_copy(rel.at[rvec], rpbs[sl], semr[sl])

        def wait(sl):
            pltpu.make_async_copy(ent.at[pl.ds(0, G)], hebs[sl], semh[sl]).wait()
            pltpu.make_async_copy(ent.at[pl.ds(0, G)], tebs[sl], semt[sl]).wait()
            pltpu.make_async_copy(relm.at[pl.ds(0, G)], mbs[sl], semm[sl]).wait()
            pltpu.make_async_copy(rel.at[pl.ds(0, G)], rpbs[sl], semr[sl]).wait()

        def compute(g, sl):
            heb, teb, mb, rpb = hebs[sl], tebs[sl], mbs[sl], rpbs[sl]
            gbase = g * G
            rowv = gbase + lanes
            flat0 = rowv * DR

            for j in range(DR):
                cj = jnp.full((L,), j, jnp.int32)
                recol = plsc.load_gather(rpb, [lanes, cj])
                plsc.store_scatter(res, [flat0 + j], recol)

            def mv_body0(i, carry):
                accs = carry[:32]
                ssh, sst = carry[32], carry[33]
                ci = jnp.full((L,), i, jnp.int32)
                hcol = plsc.load_gather(heb, [lanes, ci])
                tcol = plsc.load_gather(teb, [lanes, ci])
                colbase = i * DR
                out = []
                for j in range(16):
                    mcol = plsc.load_gather(
                        mb, [lanes, jnp.full((L,), colbase + j, jnp.int32)])
                    out.append(accs[2 * j] + hcol * mcol)
                    out.append(accs[2 * j + 1] + tcol * mcol)
                out.append(ssh + hcol * hcol)
                out.append(sst + tcol * tcol)
                return tuple(out)

            c0 = lax.fori_loop(0, DE, mv_body0, (zeros,) * 34)
            accs0, ssh, sst = c0[:32], c0[32], c0[33]
            sch = jnp.where(ssh > 1.0, _rsqrt_nr(ssh), ones)
            sct = jnp.where(sst > 1.0, _rsqrt_nr(sst), ones)
            for j in range(16):
                fj = flat0 + j
                plsc.store_scatter(hps, [fj], accs0[2 * j] * sch)
                plsc.store_scatter(tps, [fj], accs0[2 * j + 1] * sct)

            def mv_body1(i, accs):
                ci = jnp.full((L,), i, jnp.int32)
                hcol = plsc.load_gather(heb, [lanes, ci])
                tcol = plsc.load_gather(teb, [lanes, ci])
                colbase = i * DR + 16
                out = []
                for j in range(16):
                    mcol = plsc.load_gather(
                        mb, [lanes, jnp.full((L,), colbase + j, jnp.int32)])
                    out.append(accs[2 * j] + hcol * mcol)
                    out.append(accs[2 * j + 1] + tcol * mcol)
                return tuple(out)

            accs1 = lax.fori_loop(0, DE, mv_body1, (zeros,) * 32)
            for j in range(16):
                fj = flat0 + 16 + j
                plsc.store_scatter(hps, [fj], accs1[2 * j] * sch)
                plsc.store_scatter(tps, [fj], accs1[2 * j + 1] * sct)

        fire(0, 0)
        fire(1, 1)

        def epoch(k, carry):
            for sl in range(NSLOT):
                g = NSLOT * k + sl
                wait(sl)
                compute(g, sl)

                @pl.when(g + NSLOT < NG)
                def _():
                    fire(g + NSLOT, sl)
            return carry

        lax.fori_loop(0, NG // NSLOT, epoch, 0)

        pltpu.sync_copy(hps, hp.at[pl.ds(wbase * DR, BPW * DR)])
        pltpu.sync_copy(res, reo.at[pl.ds(wbase * DR, BPW * DR)])
        pltpu.sync_copy(tps, tp.at[pl.ds(wbase * DR, BPW * DR)])

    f = pl.kernel(
        body,
        out_type=(
            jax.ShapeDtypeStruct((B * DR,), jnp.float32),
            jax.ShapeDtypeStruct((B * DR,), jnp.float32),
            jax.ShapeDtypeStruct((B * DR,), jnp.float32),
        ),
        mesh=mesh,
        scratch_types=[
            pltpu.VMEM((BPW,), jnp.int32),
            pltpu.VMEM((BPW,), jnp.int32),
            pltpu.VMEM((BPW,), jnp.int32),
            pltpu.VMEM((G, DE), jnp.float32),
            pltpu.VMEM((G, DE), jnp.float32),
            pltpu.VMEM((G, DE), jnp.float32),
            pltpu.VMEM((G, DE), jnp.float32),
            pltpu.VMEM((G, DE * DR), jnp.float32),
            pltpu.VMEM((G, DE * DR), jnp.float32),
            pltpu.VMEM((G, DR), jnp.float32),
            pltpu.VMEM((G, DR), jnp.float32),
            pltpu.VMEM((BPW * DR,), jnp.float32),
            pltpu.VMEM((BPW * DR,), jnp.float32),
            pltpu.VMEM((BPW * DR,), jnp.float32),
            pltpu.SemaphoreType.DMA,
            pltpu.SemaphoreType.DMA,
            pltpu.SemaphoreType.DMA,
            pltpu.SemaphoreType.DMA,
            pltpu.SemaphoreType.DMA,
            pltpu.SemaphoreType.DMA,
            pltpu.SemaphoreType.DMA,
            pltpu.SemaphoreType.DMA,
        ],
        compiler_params=pltpu.CompilerParams(needs_layout_passes=False,
                                             use_tc_tiling_on_sc=False),
        name="transr_triplet_sc",
    )
    return f(ent, rel, relm, h, r, t)


def kernel(entityEmb, relationEmb, relationEmbM, h, r, t):
    hp, re, tp = _tripletembed(entityEmb, relationEmb, relationEmbM,
                               h.astype(jnp.int32), r.astype(jnp.int32),
                               t.astype(jnp.int32))
    return (hp.reshape(B, DR), re.reshape(B, DR), tp.reshape(B, DR))


# speedup vs baseline: 1.6711x; 1.6711x over previous
import jax
import jax.numpy as jnp
from jax import lax
from jax.experimental import pallas as pl
from jax.experimental.pallas import tpu as pltpu
from jax.experimental.pallas import tpu_sc as plsc

ENTITY_NUM = 1000000
REL_NUM = 1000
DE = 64
DR = 32
B = 16384

NC = 2
NS = 16
L = 16
NW = NC * NS
BPW = B // NW
G = L
NG = BPW // G
NSLOT = 2


def _rsqrt_nr(x):
    i = lax.bitcast_convert_type(x, jnp.int32)
    y = lax.bitcast_convert_type(jnp.int32(0x5F3759DF) - (i >> 1), jnp.float32)
    for _ in range(3):
        y = y * (1.5 - 0.5 * x * y * y)
    return y


@jax.jit
def _tripletembed(ent, rel, relm, h, r, t):
    mesh = plsc.VectorSubcoreMesh(core_axis_name="c", subcore_axis_name="s",
                                  num_cores=NC, num_subcores=NS)

    def body(ent, rel, relm, h, r, t, hp, reo, tp,
             hidx, ridx, tidx,
             heb0, heb1, teb0, teb1, mb0, mb1, rpb0, rpb1,
             hps, res, tps,
             semh0, semh1, semt0, semt1, semm0, semm1, semr0, semr1):
        c = lax.axis_index("c")
        s = lax.axis_index("s")
        wid = s * NC + c
        wbase = wid * BPW

        pltpu.sync_copy(h.at[pl.ds(wbase, BPW)], hidx)
        pltpu.sync_copy(r.at[pl.ds(wbase, BPW)], ridx)
        pltpu.sync_copy(t.at[pl.ds(wbase, BPW)], tidx)

        hebs = (heb0, heb1)
        tebs = (teb0, teb1)
        mbs = (mb0, mb1)
        rpbs = (rpb0, rpb1)
        semh = (semh0, semh1)
        semt = (semt0, semt1)
        semm = (semm0, semm1)
        semr = (semr0, semr1)

        lanes = lax.iota(jnp.int32, L)
        zeros = jnp.zeros((L,), jnp.float32)
        ones = jnp.ones((L,), jnp.float32)

        def fire(g, sl):
            gbase = g * G
            hvec = hidx[pl.ds(gbase, G)]
            tvec = tidx[pl.ds(gbase, G)]
            rvec = ridx[pl.ds(gbase, G)]
            pltpu.async_copy(ent.at[hvec], hebs[sl], semh[sl])
            pltpu.async_copy(ent.at[tvec], tebs[sl], semt[sl])
            pltpu.async_copy(relm.at[rvec], mbs[sl], semm[sl])
            pltpu.async_copy(rel.at[rvec], rpbs[sl], semr[sl])

        def wait(sl):
            pltpu.make_async_copy(ent.at[pl.ds(0, G)], hebs[sl], semh[sl]).wait()
            pltpu.make_async_copy(ent.at[pl.ds(0, G)], tebs[sl], semt[sl]).wait()
            pltpu.make_async_copy(relm.at[pl.ds(0, G)], mbs[sl], semm[sl]).wait()
            pltpu.make_async_copy(rel.at[pl.ds(0, G)], rpbs[sl], semr[sl]).wait()

        bcast_dnums = lax.GatherDimensionNumbers(
            offset_dims=(), collapsed_slice_dims=(0,), start_index_map=(0,))

        def _bcast_lane(vec, ii):
            iis = jnp.full((L, 1), ii, jnp.int32)
            return lax.gather(vec, iis, bcast_dnums, (1,),
                              mode=lax.GatherScatterMode.PROMISE_IN_BOUNDS)

        def compute(g, sl):
            heb, teb, mb, rpb = hebs[sl], tebs[sl], mbs[sl], rpbs[sl]
            gbase = g * G

            def triple(bl, carry):
                boff = (gbase + bl) * DR

                res[pl.ds(boff, 16)] = rpb[bl, pl.ds(0, 16)]
                res[pl.ds(boff + 16, 16)] = rpb[bl, pl.ds(16, 16)]

                hr = [heb[bl, pl.ds(k * 16, 16)] for k in range(4)]
                tr = [teb[bl, pl.ds(k * 16, 16)] for k in range(4)]
                sqh = ((hr[0] * hr[0] + hr[1] * hr[1])
                       + (hr[2] * hr[2] + hr[3] * hr[3]))
                sqt = ((tr[0] * tr[0] + tr[1] * tr[1])
                       + (tr[2] * tr[2] + tr[3] * tr[3]))
                ssh = jnp.full((L,), jnp.sum(sqh), jnp.float32)
                sst = jnp.full((L,), jnp.sum(sqt), jnp.float32)
                sch = jnp.where(ssh > 1.0, _rsqrt_nr(ssh), ones)
                sct = jnp.where(sst > 1.0, _rsqrt_nr(sst), ones)

                ah0 = ah1 = at0 = at1 = zeros
                for k in range(4):
                    for ii in range(16):
                        i = k * 16 + ii
                        hb = _bcast_lane(hr[k], ii)
                        tb = _bcast_lane(tr[k], ii)
                        m0 = mb[bl, pl.ds(i * DR, 16)]
                        m1 = mb[bl, pl.ds(i * DR + 16, 16)]
                        ah0 = ah0 + hb * m0
                        ah1 = ah1 + hb * m1
                        at0 = at0 + tb * m0
                        at1 = at1 + tb * m1

                hps[pl.ds(boff, 16)] = ah0 * sch
                hps[pl.ds(boff + 16, 16)] = ah1 * sch
                tps[pl.ds(boff, 16)] = at0 * sct
                tps[pl.ds(boff + 16, 16)] = at1 * sct
                return carry

            lax.fori_loop(0, G, triple, 0)

        fire(0, 0)
        fire(1, 1)

        def epoch(k, carry):
            for sl in range(NSLOT):
                g = NSLOT * k + sl
                wait(sl)
                compute(g, sl)

                @pl.when(g + NSLOT < NG)
                def _():
                    fire(g + NSLOT, sl)
            return carry

        lax.fori_loop(0, NG // NSLOT, epoch, 0)

        pltpu.sync_copy(hps, hp.at[pl.ds(wbase * DR, BPW * DR)])
        pltpu.sync_copy(res, reo.at[pl.ds(wbase * DR, BPW * DR)])
        pltpu.sync_copy(tps, tp.at[pl.ds(wbase * DR, BPW * DR)])

    f = pl.kernel(
        body,
        out_type=(
            jax.ShapeDtypeStruct((B * DR,), jnp.float32),
            jax.ShapeDtypeStruct((B * DR,), jnp.float32),
            jax.ShapeDtypeStruct((B * DR,), jnp.float32),
        ),
        mesh=mesh,
        scratch_types=[
            pltpu.VMEM((BPW,), jnp.int32),
            pltpu.VMEM((BPW,), jnp.int32),
            pltpu.VMEM((BPW,), jnp.int32),
            pltpu.VMEM((G, DE), jnp.float32),
            pltpu.VMEM((G, DE), jnp.float32),
            pltpu.VMEM((G, DE), jnp.float32),
            pltpu.VMEM((G, DE), jnp.float32),
            pltpu.VMEM((G, DE * DR), jnp.float32),
            pltpu.VMEM((G, DE * DR), jnp.float32),
            pltpu.VMEM((G, DR), jnp.float32),
            pltpu.VMEM((G, DR), jnp.float32),
            pltpu.VMEM((BPW * DR,), jnp.float32),
            pltpu.VMEM((BPW * DR,), jnp.float32),
            pltpu.VMEM((BPW * DR,), jnp.float32),
            pltpu.SemaphoreType.DMA,
            pltpu.SemaphoreType.DMA,
            pltpu.SemaphoreType.DMA,
            pltpu.SemaphoreType.DMA,
            pltpu.SemaphoreType.DMA,
            pltpu.SemaphoreType.DMA,
            pltpu.SemaphoreType.DMA,
            pltpu.SemaphoreType.DMA,
        ],
        compiler_params=pltpu.CompilerParams(needs_layout_passes=False,
                                             use_tc_tiling_on_sc=False),
        name="transr_triplet_sc",
    )
    return f(ent, rel, relm, h, r, t)


def kernel(entityEmb, relationEmb, relationEmbM, h, r, t):
    hp, re, tp = _tripletembed(entityEmb, relationEmb, relationEmbM,
                               h.astype(jnp.int32), r.astype(jnp.int32),
                               t.astype(jnp.int32))
    return (hp.reshape(B, DR), re.reshape(B, DR), tp.reshape(B, DR))
